# baseline (speedup 1.0000x reference)
import functools

import jax
import jax.numpy as jnp
from jax import lax
from jax.experimental import pallas as pl
from jax.experimental.pallas import tpu as pltpu
from jax.experimental.pallas import tpu_sc as plsc

B = 128
D = 128
M = 100000
C = 1000
CP = 1024
TEMP = 0.05

NC = 2
NS = 16
NW = NC * NS
NM = 4
ND = 8
LANES = 16
MW = M // NM
MB = 1250
NCHUNK = MW // MB
NBUF = 4
NFULL = MB // LANES
TAIL = MB - NFULL * LANES
MBP = (NFULL + 1) * LANES
MWP = MW + 2 * LANES


@functools.cache
def _make_sc_segsum():
    mesh = plsc.VectorSubcoreMesh(core_axis_name="c", subcore_axis_name="s",
                                  num_cores=NC, num_subcores=NS)
    return functools.partial(
        pl.kernel,
        out_type=(
            jax.ShapeDtypeStruct((NM, CP, D), jnp.float32),
            jax.ShapeDtypeStruct((NM, CP), jnp.float32),
            jax.ShapeDtypeStruct((B,), jnp.int32),
        ),
        mesh=mesh,
        scratch_types=[
            pltpu.VMEM((NBUF, MBP, LANES), jnp.float32),
            pltpu.VMEM((MWP,), jnp.int32),
            pltpu.VMEM((CP, LANES), jnp.float32),
            pltpu.VMEM((CP,), jnp.float32),
            pltpu.VMEM((B,), jnp.int32),
            pltpu.VMEM((B,), jnp.int32),
            pltpu.SemaphoreType.DMA,
            pltpu.SemaphoreType.DMA,
            pltpu.SemaphoreType.DMA,
            pltpu.SemaphoreType.DMA,
            pltpu.SemaphoreType.DMA,
            pltpu.SemaphoreType.DMA,
        ],
        compiler_params=pltpu.CompilerParams(use_tc_tiling_on_sc=False,
                                             needs_layout_passes=False),
    )(_sc_segsum_body)


def _sc_segsum_body(features_hbm, labels_hbm, indexes_hbm,
                    seg_out, cnt_out, tgt_out,
                    fbuf, lbuf, accv, cntv, idxv, tgtv, lsem, fsem0, fsem1,
                    fsem2, fsem3, tsem):
    cid = lax.axis_index("c")
    sid = lax.axis_index("s")
    wid = sid * NC + cid
    mrange = wid // ND
    dchunk = lax.rem(wid, ND)
    m0 = mrange * MW
    d0 = dchunk * LANES
    fsems = (fsem0, fsem1, fsem2, fsem3)

    def feat_dma(k, b):
        return pltpu.make_async_copy(
            features_hbm.at[pl.ds(m0 + k * MB, MB), pl.ds(d0, LANES)],
            fbuf.at[b, pl.ds(0, MB), :], fsems[b])

    lab_dma = pltpu.make_async_copy(labels_hbm.at[pl.ds(m0, MW)],
                                    lbuf.at[pl.ds(0, MW)], lsem)
    lab_dma.start()
    for _b in range(NBUF):
        feat_dma(_b, _b).start()

    @pl.when(wid == 1)
    def _():
        pltpu.sync_copy(indexes_hbm, idxv)
        pltpu.async_copy(labels_hbm.at[idxv], tgtv, tsem).wait()
        pltpu.sync_copy(tgtv, tgt_out)

    zero16 = jnp.zeros((LANES,), jnp.float32)

    def zero_acc(r, carry):
        accv[r, :] = zero16
        return carry

    lax.fori_loop(0, CP, zero_acc, 0)

    def zero_cnt(r, carry):
        cntv[pl.ds(r * LANES, LANES)] = zero16
        return carry

    lax.fori_loop(0, CP // LANES, zero_cnt, 0)
    lab_dma.wait()

    iota16 = lax.iota(jnp.int32, LANES)
    ones16 = jnp.ones((LANES,), jnp.float32)
    tail_msk = iota16 < TAIL
    cvecs = [jnp.bitwise_and(iota16 + cc, LANES - 1) for cc in range(LANES)]

    def do_block(fb, lbase, rowm, lv):
        vals = [plsc.load_gather(fb, [rowm, cvecs[cc]])
                for cc in range(LANES)]
        for cc in range(LANES):
            plsc.addupdate_scatter(accv, [lv, cvecs[cc]], vals[cc])

    def process_chunk(k, b):
        fb = fbuf.at[b]
        koff = k * MB

        @plsc.parallel_loop(0, NFULL * LANES, step=LANES, unroll=2)
        def block_body(base):
            rowm = base + iota16
            lv = lbuf[pl.ds(koff + base, LANES)]
            do_block(fb, koff + base, rowm, lv)

            @pl.when(dchunk == 0)
            def _():
                plsc.addupdate_scatter(cntv, [lv], ones16)
        base = NFULL * LANES
        rowm = base + iota16
        lv = jnp.where(tail_msk, lbuf[pl.ds(koff + base, LANES)], 0)
        for cc in range(LANES):
            cvec = jnp.bitwise_and(iota16 + cc, LANES - 1)
            vals = plsc.load_gather(fb, [rowm, cvec], mask=tail_msk)
            plsc.addupdate_scatter(accv, [lv, cvec], vals, mask=tail_msk)

        @pl.when(dchunk == 0)
        def _():
            plsc.addupdate_scatter(cntv, [lv], ones16, mask=tail_msk)

    def ring_body(i, carry):
        for b in range(NBUF):
            k = NBUF * i + b
            feat_dma(k, b).wait()
            process_chunk(k, b)

            @pl.when(i < NCHUNK // NBUF - 1)
            def _():
                feat_dma(k + NBUF, b).start()

        return carry

    lax.fori_loop(0, NCHUNK // NBUF, ring_body, 0)

    pltpu.sync_copy(accv, seg_out.at[mrange, :, pl.ds(d0, LANES)])

    @pl.when(dchunk == 0)
    def _():
        pltpu.sync_copy(cntv, cnt_out.at[mrange])


def _tc_body(feat_ref, gen_ref, aug_ref, seg_ref, cnt_ref, tcol_ref, trow_ref,
             out_ref):
    inv_temp = 1.0 / TEMP

    def norm(x):
        n = jnp.sqrt(jnp.sum(x * x, axis=1, keepdims=True))
        return x / jnp.maximum(n, 1e-12)

    def dot_t(a, b):
        return lax.dot_general(a, b, (((1,), (1,)), ((), ())),
                               precision=lax.Precision.HIGHEST,
                               preferred_element_type=jnp.float32)

    ori = norm(feat_ref[...])
    gen = norm(gen_ref[...])
    aug = norm(aug_ref[...])
    sim_ori = dot_t(ori, ori) * inv_temp
    sim_gen = dot_t(ori, gen) * inv_temp
    sim_aug = dot_t(ori, aug) * inv_temp

    seg = jnp.sum(seg_ref[...], axis=0)
    cnt = jnp.sum(cnt_ref[...], axis=0, keepdims=True)
    mask = (cnt > 0).astype(jnp.float32)
    sim_t = dot_t(aug, seg) * inv_temp
    denom = mask * cnt + (1.0 - mask)
    sim_t = sim_t / denom
    exps = jnp.exp(sim_t) * mask
    sums = jnp.sum(exps, axis=1, keepdims=True) + 1e-6
    msim = exps / sums

    tcol = tcol_ref[...]
    cols = lax.broadcasted_iota(jnp.int32, (B, CP), 1)
    oneh = cols == tcol
    pick = jnp.sum(jnp.where(oneh, msim, 0.0), axis=1, keepdims=True)
    spcl = -jnp.mean(jnp.log(pick + 1e-6))

    trow = trow_ref[...]
    same = tcol == trow
    diff = jnp.logical_not(same).astype(jnp.float32)
    exp_ori = jnp.exp(sim_ori)
    exp_gen = jnp.exp(sim_gen)
    exp_aug = jnp.exp(sim_aug)
    gsum = jnp.sum(exp_gen, axis=1, keepdims=True)
    exp_sum_co = jnp.sum(diff * (exp_ori + exp_aug), axis=1, keepdims=True)
    denom_co = exp_sum_co + gsum + 1e-6
    term_o = -jnp.log(exp_ori / (exp_ori + denom_co) + 1e-6)
    term_a = -jnp.log(exp_aug / (exp_aug + denom_co) + 1e-6)
    co = jnp.sum(jnp.where(same, term_o + term_a, 0.0)) / B
    exp_sum_ad = jnp.sum(diff * exp_gen, axis=1, keepdims=True)
    denom_ad = (exp_sum_ad + jnp.sum(exp_aug, axis=1, keepdims=True)
                + jnp.sum(exp_ori, axis=1, keepdims=True) + 1e-6)
    term_g = -jnp.log(exp_gen / (exp_gen + denom_ad) + 1e-6)
    ad = jnp.sum(jnp.where(same, term_g, 0.0)) / B

    li = lax.broadcasted_iota(jnp.int32, (1, B), 1)
    out_ref[...] = jnp.where(
        li == 0, spcl, jnp.where(li == 1, ad, jnp.where(li == 2, co, 0.0)))


_tc_call = pl.pallas_call(
    _tc_body,
    out_shape=jax.ShapeDtypeStruct((1, B), jnp.float32),
)


def kernel(feat, feat_gen, feat_aug, indexes, features, labels):
    seg_parts, cnt_parts, targets = _make_sc_segsum()(
        features, labels, indexes)
    tcol = targets.reshape(B, 1)
    trow = targets.reshape(1, B)
    o = _tc_call(feat, feat_gen, feat_aug, seg_parts, cnt_parts, tcol, trow)
    return (o[0, 0], o[0, 1], o[0, 2])


# speedup vs baseline: 1.0427x; 1.0427x over previous
import functools

import jax
import jax.numpy as jnp
from jax import lax
from jax.experimental import pallas as pl
from jax.experimental.pallas import tpu as pltpu
from jax.experimental.pallas import tpu_sc as plsc

B = 128
D = 128
M = 100000
C = 1000
CP = 1024
TEMP = 0.05

NC = 2
NS = 16
NW = NC * NS
NM = 4
ND = 8
LANES = 16
MW = M // NM
MB = 2500
NCHUNK = MW // MB
NFULL = MB // LANES
TAIL = MB - NFULL * LANES
MBP = (NFULL + 1) * LANES
MWP = MW + 2 * LANES


@functools.cache
def _make_sc_segsum():
    mesh = plsc.VectorSubcoreMesh(core_axis_name="c", subcore_axis_name="s",
                                  num_cores=NC, num_subcores=NS)
    return functools.partial(
        pl.kernel,
        out_type=(
            jax.ShapeDtypeStruct((NM, CP, D), jnp.float32),
            jax.ShapeDtypeStruct((NM, CP), jnp.float32),
            jax.ShapeDtypeStruct((B,), jnp.int32),
        ),
        mesh=mesh,
        scratch_types=[
            pltpu.VMEM((2, MBP, LANES), jnp.float32),
            pltpu.VMEM((MWP,), jnp.int32),
            pltpu.VMEM((CP, LANES), jnp.float32),
            pltpu.VMEM((CP,), jnp.float32),
            pltpu.VMEM((B,), jnp.int32),
            pltpu.VMEM((B,), jnp.int32),
            pltpu.SemaphoreType.DMA,
            pltpu.SemaphoreType.DMA,
            pltpu.SemaphoreType.DMA,
            pltpu.SemaphoreType.DMA,
        ],
        compiler_params=pltpu.CompilerParams(use_tc_tiling_on_sc=False,
                                             needs_layout_passes=False),
    )(_sc_segsum_body)


def _sc_segsum_body(features_hbm, labels_hbm, indexes_hbm,
                    seg_out, cnt_out, tgt_out,
                    fbuf, lbuf, accv, cntv, idxv, tgtv, lsem, fsem0, fsem1,
                    tsem):
    cid = lax.axis_index("c")
    sid = lax.axis_index("s")
    wid = sid * NC + cid
    mrange = wid // ND
    dchunk = lax.rem(wid, ND)
    m0 = mrange * MW
    d0 = dchunk * LANES
    fsems = (fsem0, fsem1)

    def feat_dma(k, b):
        return pltpu.make_async_copy(
            features_hbm.at[pl.ds(m0 + k * MB, MB), pl.ds(d0, LANES)],
            fbuf.at[b, pl.ds(0, MB), :], fsems[b])

    lab_dma = pltpu.make_async_copy(labels_hbm.at[pl.ds(m0, MW)],
                                    lbuf.at[pl.ds(0, MW)], lsem)
    lab_dma.start()
    feat_dma(0, 0).start()
    feat_dma(1, 1).start()

    @pl.when(wid == 1)
    def _():
        pltpu.sync_copy(indexes_hbm, idxv)
        pltpu.async_copy(labels_hbm.at[idxv], tgtv, tsem).wait()
        pltpu.sync_copy(tgtv, tgt_out)

    zero16 = jnp.zeros((LANES,), jnp.float32)

    def zero_acc(r, carry):
        accv[r, :] = zero16
        return carry

    lax.fori_loop(0, CP, zero_acc, 0)

    def zero_cnt(r, carry):
        cntv[pl.ds(r * LANES, LANES)] = zero16
        return carry

    lax.fori_loop(0, CP // LANES, zero_cnt, 0)
    lab_dma.wait()

    iota16 = lax.iota(jnp.int32, LANES)
    ones16 = jnp.ones((LANES,), jnp.float32)
    tail_msk = iota16 < TAIL
    cvecs = [jnp.bitwise_and(iota16 + cc, LANES - 1) for cc in range(LANES)]

    def do_block(fb, lbase, rowm, lv):
        vals = [plsc.load_gather(fb, [rowm, cvecs[cc]])
                for cc in range(LANES)]
        for cc in range(LANES):
            plsc.addupdate_scatter(accv, [lv, cvecs[cc]], vals[cc])

    def process_chunk(k, b):
        fb = fbuf.at[b]
        koff = k * MB

        @plsc.parallel_loop(0, NFULL * LANES, step=LANES, unroll=2)
        def block_body(base):
            rowm = base + iota16
            lv = lbuf[pl.ds(koff + base, LANES)]
            do_block(fb, koff + base, rowm, lv)

            @pl.when(dchunk == 0)
            def _():
                plsc.addupdate_scatter(cntv, [lv], ones16)
        base = NFULL * LANES
        rowm = base + iota16
        lv = jnp.where(tail_msk, lbuf[pl.ds(koff + base, LANES)], 0)
        for cc in range(LANES):
            cvec = jnp.bitwise_and(iota16 + cc, LANES - 1)
            vals = plsc.load_gather(fb, [rowm, cvec], mask=tail_msk)
            plsc.addupdate_scatter(accv, [lv, cvec], vals, mask=tail_msk)

        @pl.when(dchunk == 0)
        def _():
            plsc.addupdate_scatter(cntv, [lv], ones16, mask=tail_msk)

    def pair_body(i, carry):
        for b in range(2):
            k = 2 * i + b
            feat_dma(k, b).wait()
            process_chunk(k, b)

            @pl.when(i < NCHUNK // 2 - 1)
            def _():
                feat_dma(k + 2, b).start()

        return carry

    lax.fori_loop(0, NCHUNK // 2, pair_body, 0)

    pltpu.sync_copy(accv, seg_out.at[mrange, :, pl.ds(d0, LANES)])

    @pl.when(dchunk == 0)
    def _():
        pltpu.sync_copy(cntv, cnt_out.at[mrange])


def _tc_body(feat_ref, gen_ref, aug_ref, seg_ref, cnt_ref, tcol_ref, trow_ref,
             out_ref):
    inv_temp = 1.0 / TEMP

    def norm(x):
        n = jnp.sqrt(jnp.sum(x * x, axis=1, keepdims=True))
        return x / jnp.maximum(n, 1e-12)

    def dot_t(a, b):
        return lax.dot_general(a, b, (((1,), (1,)), ((), ())),
                               precision=lax.Precision.HIGHEST,
                               preferred_element_type=jnp.float32)

    ori = norm(feat_ref[...])
    gen = norm(gen_ref[...])
    aug = norm(aug_ref[...])
    sim_ori = dot_t(ori, ori) * inv_temp
    sim_gen = dot_t(ori, gen) * inv_temp
    sim_aug = dot_t(ori, aug) * inv_temp

    seg = jnp.sum(seg_ref[...], axis=0)
    cnt = jnp.sum(cnt_ref[...], axis=0, keepdims=True)
    mask = (cnt > 0).astype(jnp.float32)
    sim_t = dot_t(aug, seg) * inv_temp
    denom = mask * cnt + (1.0 - mask)
    sim_t = sim_t / denom
    exps = jnp.exp(sim_t) * mask
    sums = jnp.sum(exps, axis=1, keepdims=True) + 1e-6
    msim = exps / sums

    tcol = tcol_ref[...]
    cols = lax.broadcasted_iota(jnp.int32, (B, CP), 1)
    oneh = cols == tcol
    pick = jnp.sum(jnp.where(oneh, msim, 0.0), axis=1, keepdims=True)
    spcl = -jnp.mean(jnp.log(pick + 1e-6))

    trow = trow_ref[...]
    same = tcol == trow
    diff = jnp.logical_not(same).astype(jnp.float32)
    exp_ori = jnp.exp(sim_ori)
    exp_gen = jnp.exp(sim_gen)
    exp_aug = jnp.exp(sim_aug)
    gsum = jnp.sum(exp_gen, axis=1, keepdims=True)
    exp_sum_co = jnp.sum(diff * (exp_ori + exp_aug), axis=1, keepdims=True)
    denom_co = exp_sum_co + gsum + 1e-6
    term_o = -jnp.log(exp_ori / (exp_ori + denom_co) + 1e-6)
    term_a = -jnp.log(exp_aug / (exp_aug + denom_co) + 1e-6)
    co = jnp.sum(jnp.where(same, term_o + term_a, 0.0)) / B
    exp_sum_ad = jnp.sum(diff * exp_gen, axis=1, keepdims=True)
    denom_ad = (exp_sum_ad + jnp.sum(exp_aug, axis=1, keepdims=True)
                + jnp.sum(exp_ori, axis=1, keepdims=True) + 1e-6)
    term_g = -jnp.log(exp_gen / (exp_gen + denom_ad) + 1e-6)
    ad = jnp.sum(jnp.where(same, term_g, 0.0)) / B

    li = lax.broadcasted_iota(jnp.int32, (1, B), 1)
    out_ref[...] = jnp.where(
        li == 0, spcl, jnp.where(li == 1, ad, jnp.where(li == 2, co, 0.0)))


_tc_call = pl.pallas_call(
    _tc_body,
    out_shape=jax.ShapeDtypeStruct((1, B), jnp.float32),
)


def kernel(feat, feat_gen, feat_aug, indexes, features, labels):
    seg_parts, cnt_parts, targets = _make_sc_segsum()(
        features, labels, indexes)
    tcol = targets.reshape(B, 1)
    trow = targets.reshape(1, B)
    o = _tc_call(feat, feat_gen, feat_aug, seg_parts, cnt_parts, tcol, trow)
    return (o[0, 0], o[0, 1], o[0, 2])
